# Initial kernel scaffold; baseline (speedup 1.0000x reference)
import jax
import jax.numpy as jnp
from jax.experimental import pallas as pl


def kernel(item_indices, context_indices, item_table, set_table, W1, b1, W2, b2, W3, b3):
    raise NotImplementedError("write your pallas kernel here")


# speedup vs baseline: 1.4632x; 1.4632x over previous
import functools

import jax
import jax.numpy as jnp
from jax import lax
from jax.experimental import pallas as pl
from jax.experimental.pallas import tpu as pltpu
from jax.experimental.pallas import tpu_sc as plsc

NUM_ITEMS = 1000000
EMBED = 32
CTX = 32
B = 4096
L_ITEM = 20
L_CTX = 200

NC = 2
NS = 16
NW = NC * NS
BPW = B // NW
CB = 8
N_CCHUNK = BPW // CB
CROWS = CB * L_CTX
IROWS_PER_W = BPW * L_ITEM
ICHUNK = 512
N_ICHUNK = IROWS_PER_W // ICHUNK

_SELU_ALPHA = 1.6732632423543772
_SELU_SCALE = 1.0507009873554805


def _sc_body(item_idx_hbm, ctx_idx_hbm, item_tab_hbm, set_tab_hbm,
             summed_hbm, irows_out_hbm,
             cidx_v, crows_v, acc_v, iidx_v, irows_v, sem):
    c = lax.axis_index("c")
    s = lax.axis_index("s")
    w = s * NC + c
    cbase = w * (BPW * L_CTX)
    ibase = w * IROWS_PER_W

    pltpu.sync_copy(item_idx_hbm.at[pl.ds(pl.multiple_of(ibase, 8), IROWS_PER_W)],
                    iidx_v)

    def item_chunk(k, _):
        off = pl.multiple_of(ibase + k * ICHUNK, 8)
        pltpu.async_copy(item_tab_hbm.at[iidx_v.at[pl.ds(k * ICHUNK, ICHUNK)]],
                         irows_v, sem).wait()
        pltpu.sync_copy(irows_v, irows_out_hbm.at[pl.ds(off, ICHUNK)])
        return 0

    lax.fori_loop(0, N_ICHUNK, item_chunk, 0)

    def ctx_chunk(g, _):
        goff = pl.multiple_of(cbase + g * CROWS, 8)
        pltpu.sync_copy(ctx_idx_hbm.at[pl.ds(goff, CROWS)], cidx_v)
        pltpu.async_copy(set_tab_hbm.at[cidx_v], crows_v, sem).wait()
        for r in range(CB):
            z = jnp.zeros((16,), jnp.float32)

            def body(j, carry, r=r):
                a0, a1 = carry
                row = r * L_CTX + j
                a0 = a0 + crows_v[row, pl.ds(0, 16)]
                a1 = a1 + crows_v[row, pl.ds(16, 16)]
                return (a0, a1)

            a0, a1 = plsc.parallel_loop(0, L_CTX, unroll=8, carry=(z, z))(body)
            acc_v[g * CB + r, pl.ds(0, 16)] = a0
            acc_v[g * CB + r, pl.ds(16, 16)] = a1
        return 0

    lax.fori_loop(0, N_CCHUNK, ctx_chunk, 0)

    pltpu.sync_copy(acc_v, summed_hbm.at[pl.ds(pl.multiple_of(w * BPW, 8), BPW)])


def _sc_gather_pool(item_idx, ctx_idx, item_table, set_table):
    mesh = plsc.VectorSubcoreMesh(core_axis_name="c", subcore_axis_name="s")
    return pl.kernel(
        _sc_body,
        out_type=[
            jax.ShapeDtypeStruct((B, EMBED), jnp.float32),
            jax.ShapeDtypeStruct((B * L_ITEM, EMBED), jnp.float32),
        ],
        mesh=mesh,
        scratch_types=[
            pltpu.VMEM((CROWS,), jnp.int32),
            pltpu.VMEM((CROWS, EMBED), jnp.float32),
            pltpu.VMEM((BPW, EMBED), jnp.float32),
            pltpu.VMEM((IROWS_PER_W,), jnp.int32),
            pltpu.VMEM((ICHUNK, EMBED), jnp.float32),
            pltpu.SemaphoreType.DMA,
        ],
        compiler_params=pltpu.CompilerParams(use_tc_tiling_on_sc=False),
    )(item_idx, ctx_idx, item_table, set_table)


def _selu(x):
    return _SELU_SCALE * jnp.where(x > 0, x, _SELU_ALPHA * (jnp.exp(x) - 1.0))


def _tc_body(summed_ref, ctx_idx_ref, st0_ref, irows_ref,
             W1_ref, b1_ref, W2_ref, b2_ref, W3_ref, b3_ref, out_ref):
    summed = summed_ref[...]
    idx = ctx_idx_ref[...]
    zcnt = jnp.sum(jnp.where(idx == 0, 1.0, 0.0).astype(jnp.float32),
                   axis=1, keepdims=True)
    s = summed - zcnt * st0_ref[...]
    sq = jnp.sum(s * s, axis=-1, keepdims=True)
    n = s * lax.rsqrt(jnp.maximum(sq, 1e-4))
    h = _selu(jnp.dot(n, W1_ref[...], preferred_element_type=jnp.float32)
              + b1_ref[...])
    h = _selu(jnp.dot(h, W2_ref[...], preferred_element_type=jnp.float32)
              + b2_ref[...])
    ce = (jnp.dot(h, W3_ref[...], preferred_element_type=jnp.float32)
          + b3_ref[...])
    items = irows_ref[...]
    cet = jnp.concatenate([ce] * L_ITEM, axis=1)
    d2 = (items - cet) * (items - cet)
    rows = lax.broadcasted_iota(jnp.int32, (L_ITEM * EMBED, L_ITEM), 0)
    cols = lax.broadcasted_iota(jnp.int32, (L_ITEM * EMBED, L_ITEM), 1)
    G = jnp.where(rows // EMBED == cols, 1.0, 0.0).astype(jnp.float32)
    d = jnp.dot(d2, G, preferred_element_type=jnp.float32)
    out_ref[...] = 1.0 / (1.0 + d)


def _tc_tail(summed, ctx_idx, st0, irows, W1, b1, W2, b2, W3, b3):
    BT = 256
    grid = (B // BT,)
    full = lambda shape: pl.BlockSpec(shape, lambda i: (0, 0))
    return pl.pallas_call(
        _tc_body,
        grid=grid,
        in_specs=[
            pl.BlockSpec((BT, EMBED), lambda i: (i, 0)),
            pl.BlockSpec((BT, L_CTX), lambda i: (i, 0)),
            full((1, EMBED)),
            pl.BlockSpec((BT, L_ITEM * EMBED), lambda i: (i, 0)),
            full((CTX, 2 * CTX)),
            full((1, 2 * CTX)),
            full((2 * CTX, 4 * CTX)),
            full((1, 4 * CTX)),
            full((4 * CTX, EMBED)),
            full((1, EMBED)),
        ],
        out_specs=pl.BlockSpec((BT, L_ITEM), lambda i: (i, 0)),
        out_shape=jax.ShapeDtypeStruct((B, L_ITEM), jnp.float32),
    )(summed, ctx_idx, st0, irows, W1, b1, W2, b2, W3, b3)


def kernel(item_indices, context_indices, item_table, set_table,
           W1, b1, W2, b2, W3, b3):
    ctx_i32 = context_indices.astype(jnp.int32)
    ctx_adj = jnp.maximum(ctx_i32 - 1, 0).reshape(-1)
    iidx = item_indices.astype(jnp.int32).reshape(-1)
    summed, irows = _sc_gather_pool(iidx, ctx_adj, item_table, set_table)
    out = _tc_tail(summed, ctx_i32, set_table[0:1],
                   irows.reshape(B, L_ITEM * EMBED),
                   W1, b1.reshape(1, -1), W2, b2.reshape(1, -1),
                   W3, b3.reshape(1, -1))
    return out
